# baseline (speedup 1.0000x reference)
import functools

import jax
import jax.numpy as jnp
from jax import lax
from jax.experimental import pallas as pl
from jax.experimental.pallas import tpu as pltpu
from jax.experimental.pallas import tpu_sc as plsc

_NC = 2
_NS = 16
_B = 100

_f32 = jnp.float32


def _pad_rows(N):
    return ((N + 8 * _NS - 1) // (8 * _NS)) * (8 * _NS)


_BD = 100


@functools.lru_cache(maxsize=None)
def _make_deg(N, E):
    EPT = E // (_NC * _NS)
    ITERS = EPT // _BD
    GROUPS = ITERS // _NBUF
    NP = _pad_rows(N)
    RPT = NP // _NS
    mesh = plsc.VectorSubcoreMesh(core_axis_name="c", subcore_axis_name="s")

    @functools.partial(
        pl.kernel,
        out_type=jax.ShapeDtypeStruct((_NC * NP, 8), _f32),
        mesh=mesh,
        scratch_types=[
            pltpu.VMEM((ITERS, _BD), jnp.int32),
            pltpu.VMEM((_BD, 8), _f32),
            pltpu.VMEM_SHARED((NP, 8), _f32),
        ] + [pltpu.SemaphoreType.DMA] * _NBUF,
        compiler_params=pltpu.CompilerParams(use_tc_tiling_on_sc=False),
    )
    def deg_kernel(ei3_hbm, ones_hbm, zeros_hbm, out_hbm, idxd, rows, acc,
                   *sems):
        c = lax.axis_index("c")
        s = lax.axis_index("s")
        rowbase = (c * _NS + s) * ITERS
        pltpu.sync_copy(zeros_hbm.at[pl.ds(s * RPT, RPT)],
                        acc.at[pl.ds(s * RPT, RPT)])
        pltpu.sync_copy(ones_hbm, rows)
        pltpu.sync_copy(ei3_hbm.at[1, pl.ds(rowbase, ITERS)], idxd)
        plsc.subcore_barrier()

        for b in range(_NBUF):
            pltpu.async_copy(rows, acc.at[idxd.at[b]], sems[b], add=True)

        def group(g, carry):
            for b in range(_NBUF):
                chunk = g * _NBUF + b
                pltpu.make_async_copy(rows, acc.at[idxd.at[chunk]],
                                      sems[b]).wait()

                @pl.when(g < GROUPS - 1)
                def _next():
                    pltpu.async_copy(rows, acc.at[idxd.at[chunk + _NBUF]],
                                     sems[b], add=True)
            return carry

        lax.fori_loop(0, GROUPS, group, 0)
        plsc.subcore_barrier()
        pltpu.sync_copy(acc.at[pl.ds(s * RPT, RPT)],
                        out_hbm.at[pl.ds(c * NP + s * RPT, RPT)])

    return deg_kernel


_NBUF = 5


@functools.lru_cache(maxsize=None)
def _make_agg(N, E, F, nbuf):
    EPT = E // (_NC * _NS)
    ITERS = EPT // _B
    GROUPS = ITERS // nbuf
    NP = _pad_rows(N)
    RPT = NP // _NS
    mesh = plsc.VectorSubcoreMesh(core_axis_name="c", subcore_axis_name="s")

    @functools.partial(
        pl.kernel,
        out_type=jax.ShapeDtypeStruct((_NC * NP, F), _f32),
        mesh=mesh,
        scratch_types=[
            pltpu.VMEM((ITERS, _B), jnp.int32),
            pltpu.VMEM((ITERS, _B), jnp.int32),
            pltpu.VMEM((nbuf, _B, F), _f32),
            pltpu.VMEM_SHARED((NP, F), _f32),
        ] + [pltpu.SemaphoreType.DMA] * nbuf,
        compiler_params=pltpu.CompilerParams(use_tc_tiling_on_sc=False),
    )
    def agg_kernel(ei3_hbm, table_hbm, zeros_hbm, out_hbm,
                   idxs, idxd, bufs, acc, *sems):
        c = lax.axis_index("c")
        s = lax.axis_index("s")
        rowbase = (c * _NS + s) * ITERS
        pltpu.sync_copy(zeros_hbm.at[pl.ds(s * RPT, RPT)],
                        acc.at[pl.ds(s * RPT, RPT)])
        pltpu.sync_copy(ei3_hbm.at[0, pl.ds(rowbase, ITERS)], idxs)
        pltpu.sync_copy(ei3_hbm.at[1, pl.ds(rowbase, ITERS)], idxd)
        plsc.subcore_barrier()

        for b in range(nbuf):
            pltpu.async_copy(table_hbm.at[idxs.at[b]], bufs.at[b], sems[b])

        def group(g, carry):
            for b in range(nbuf):
                chunk = g * nbuf + b
                pltpu.make_async_copy(table_hbm.at[idxs.at[chunk]],
                                      bufs.at[b], sems[b]).wait()
                pltpu.sync_copy(bufs.at[b], acc.at[idxd.at[chunk]],
                                add=True)

                @pl.when(g < GROUPS - 1)
                def _prefetch():
                    pltpu.async_copy(table_hbm.at[idxs.at[chunk + nbuf]],
                                     bufs.at[b], sems[b])
            return carry

        lax.fori_loop(0, GROUPS, group, 0)
        plsc.subcore_barrier()
        pltpu.sync_copy(acc.at[pl.ds(s * RPT, RPT)],
                        out_hbm.at[pl.ds(c * NP + s * RPT, RPT)])

    return agg_kernel


def _tc_bn(x, *, interpret=False):
    N, D = x.shape

    def body(x_ref, hb_ref):
        xv = x_ref[...]
        mean = jnp.mean(xv, axis=0, keepdims=True)
        var = jnp.mean((xv - mean) ** 2, axis=0, keepdims=True)
        hb_ref[...] = (xv - mean) / jnp.sqrt(var + 1e-5)

    return pl.pallas_call(
        body,
        out_shape=jax.ShapeDtypeStruct((N, D), _f32),
        interpret=interpret,
    )(x)


def _tc_scale(h_bn, deg_raw, *, interpret=False):
    N, D = h_bn.shape
    NP = _pad_rows(N)

    def body(hb_ref, dr_ref, hp_ref, dis_ref):
        dr = dr_ref[...]
        deg = (jnp.sum(dr[:N], axis=1, keepdims=True)
               + jnp.sum(dr[NP:NP + N], axis=1, keepdims=True)) / 8.0
        dis = 1.0 / jnp.sqrt(jnp.maximum(deg, 1.0))
        dis_ref[...] = dis
        hp_ref[...] = hb_ref[...] * dis

    return pl.pallas_call(
        body,
        out_shape=[jax.ShapeDtypeStruct((N, D), _f32),
                   jax.ShapeDtypeStruct((N, 1), _f32)],
        interpret=interpret,
    )(h_bn, deg_raw)


def _tc_mid(agg_raw, dis, W1, b1, W2p, *, interpret=False):
    N = dis.shape[0]
    NP = _pad_rows(N)
    CP = W2p.shape[1]

    def body(ar_ref, dis_ref, w1_ref, b1_ref, w2_ref, u_ref):
        a = ar_ref[...]
        d = dis_ref[...]
        agg = (a[:N] + a[NP:NP + N]) * d
        t = jnp.dot(agg, w1_ref[...], preferred_element_type=_f32,
                    precision=lax.Precision.HIGHEST)
        t = t + b1_ref[...][None, :]
        t = t - jnp.mean(t, axis=0, keepdims=True)
        rn = jnp.sqrt(1e-6 + jnp.mean(jnp.sum(t * t, axis=1)))
        t = jnp.maximum(t / rn, 0.0) * d
        u_ref[...] = jnp.dot(t, w2_ref[...], preferred_element_type=_f32,
                            precision=lax.Precision.HIGHEST)

    return pl.pallas_call(
        body,
        out_shape=jax.ShapeDtypeStruct((N, CP), _f32),
        interpret=interpret,
    )(agg_raw, dis, W1, b1, W2p)


def _tc_out(agg2_raw, dis, b2, *, interpret=False):
    N = dis.shape[0]
    NP = _pad_rows(N)
    C = b2.shape[0]

    def body(ar_ref, dis_ref, b2_ref, o_ref):
        a = ar_ref[...]
        v = (a[:N] + a[NP:NP + N])[:, :C] * dis_ref[...] + b2_ref[...][None, :]
        v = v - jnp.mean(v, axis=0, keepdims=True)
        rn = jnp.sqrt(1e-6 + jnp.mean(jnp.sum(v * v, axis=1)))
        o_ref[...] = v / rn

    return pl.pallas_call(
        body,
        out_shape=jax.ShapeDtypeStruct((N, C), _f32),
        interpret=interpret,
    )(agg2_raw, dis, b2)


def kernel(x, edge_index, W1, b1, W2, b2):
    N, D = x.shape
    E = edge_index.shape[1]
    C = W2.shape[1]
    CP = 48

    ei3 = edge_index.reshape(2, E // _B, _B)

    NP = _pad_rows(N)
    deg_raw = _make_deg(N, E)(
        ei3, jnp.ones((_BD, 8), _f32), jnp.zeros((NP, 8), _f32))
    h_bn = _tc_bn(x)
    h_pre, dis = _tc_scale(h_bn, deg_raw)
    agg1 = _make_agg(N, E, D, 2)(ei3, h_pre, jnp.zeros((NP, D), _f32))
    W2p = jnp.pad(W2, ((0, 0), (0, CP - C)))
    u_pad = _tc_mid(agg1, dis, W1, b1, W2p)
    agg2 = _make_agg(N, E, CP, 3)(ei3, u_pad, jnp.zeros((NP, CP), _f32))
    return _tc_out(agg2, dis, b2)


# speedup vs baseline: 1.1266x; 1.1266x over previous
import functools

import jax
import jax.numpy as jnp
from jax import lax
from jax.experimental import pallas as pl
from jax.experimental.pallas import tpu as pltpu
from jax.experimental.pallas import tpu_sc as plsc

_NC = 2
_NS = 16
_B = 40

_f32 = jnp.float32


def _pad_rows(N):
    return ((N + 8 * _NS - 1) // (8 * _NS)) * (8 * _NS)


_BD = 40


@functools.lru_cache(maxsize=None)
def _make_deg(N, E):
    EPT = E // (_NC * _NS)
    ITERS = EPT // _BD
    GROUPS = ITERS // _NBUF
    NP = _pad_rows(N)
    RPT = NP // _NS
    mesh = plsc.VectorSubcoreMesh(core_axis_name="c", subcore_axis_name="s")

    @functools.partial(
        pl.kernel,
        out_type=jax.ShapeDtypeStruct((_NC * NP, 8), _f32),
        mesh=mesh,
        scratch_types=[
            pltpu.VMEM((ITERS, _BD), jnp.int32),
            pltpu.VMEM((_BD, 8), _f32),
            pltpu.VMEM_SHARED((NP, 8), _f32),
        ] + [pltpu.SemaphoreType.DMA] * _NBUF,
        compiler_params=pltpu.CompilerParams(use_tc_tiling_on_sc=False),
    )
    def deg_kernel(ei3_hbm, ones_hbm, zeros_hbm, out_hbm, idxd, rows, acc,
                   *sems):
        c = lax.axis_index("c")
        s = lax.axis_index("s")
        rowbase = (c * _NS + s) * ITERS
        pltpu.sync_copy(zeros_hbm.at[pl.ds(s * RPT, RPT)],
                        acc.at[pl.ds(s * RPT, RPT)])
        pltpu.sync_copy(ones_hbm, rows)
        pltpu.sync_copy(ei3_hbm.at[1, pl.ds(rowbase, ITERS)], idxd)
        plsc.subcore_barrier()

        for b in range(_NBUF):
            pltpu.async_copy(rows, acc.at[idxd.at[b]], sems[b], add=True)

        def group(g, carry):
            for b in range(_NBUF):
                chunk = g * _NBUF + b
                pltpu.make_async_copy(rows, acc.at[idxd.at[chunk]],
                                      sems[b]).wait()

                @pl.when(g < GROUPS - 1)
                def _next():
                    pltpu.async_copy(rows, acc.at[idxd.at[chunk + _NBUF]],
                                     sems[b], add=True)
            return carry

        lax.fori_loop(0, GROUPS, group, 0)
        plsc.subcore_barrier()
        pltpu.sync_copy(acc.at[pl.ds(s * RPT, RPT)],
                        out_hbm.at[pl.ds(c * NP + s * RPT, RPT)])

    return deg_kernel


_NBUF = 5


@functools.lru_cache(maxsize=None)
def _make_agg(N, E, F, nbuf):
    EPT = E // (_NC * _NS)
    ITERS = EPT // _B
    GROUPS = ITERS // nbuf
    NP = _pad_rows(N)
    RPT = NP // _NS
    mesh = plsc.VectorSubcoreMesh(core_axis_name="c", subcore_axis_name="s")

    @functools.partial(
        pl.kernel,
        out_type=jax.ShapeDtypeStruct((_NC * NP, F), _f32),
        mesh=mesh,
        scratch_types=[
            pltpu.VMEM((ITERS, _B), jnp.int32),
            pltpu.VMEM((ITERS, _B), jnp.int32),
            pltpu.VMEM((nbuf, _B, F), _f32),
            pltpu.VMEM_SHARED((NP, F), _f32),
        ] + [pltpu.SemaphoreType.DMA] * nbuf,
        compiler_params=pltpu.CompilerParams(use_tc_tiling_on_sc=False),
    )
    def agg_kernel(ei3_hbm, table_hbm, zeros_hbm, out_hbm,
                   idxs, idxd, bufs, acc, *sems):
        c = lax.axis_index("c")
        s = lax.axis_index("s")
        rowbase = (c * _NS + s) * ITERS
        pltpu.sync_copy(zeros_hbm.at[pl.ds(s * RPT, RPT)],
                        acc.at[pl.ds(s * RPT, RPT)])
        pltpu.sync_copy(ei3_hbm.at[0, pl.ds(rowbase, ITERS)], idxs)
        pltpu.sync_copy(ei3_hbm.at[1, pl.ds(rowbase, ITERS)], idxd)
        plsc.subcore_barrier()

        for b in range(nbuf):
            pltpu.async_copy(table_hbm.at[idxs.at[b]], bufs.at[b], sems[b])

        def group(g, carry):
            for b in range(nbuf):
                chunk = g * nbuf + b
                pltpu.make_async_copy(table_hbm.at[idxs.at[chunk]],
                                      bufs.at[b], sems[b]).wait()
                pltpu.sync_copy(bufs.at[b], acc.at[idxd.at[chunk]],
                                add=True)

                @pl.when(g < GROUPS - 1)
                def _prefetch():
                    pltpu.async_copy(table_hbm.at[idxs.at[chunk + nbuf]],
                                     bufs.at[b], sems[b])
            return carry

        lax.fori_loop(0, GROUPS, group, 0)
        plsc.subcore_barrier()
        pltpu.sync_copy(acc.at[pl.ds(s * RPT, RPT)],
                        out_hbm.at[pl.ds(c * NP + s * RPT, RPT)])

    return agg_kernel


def _tc_bn(x, *, interpret=False):
    N, D = x.shape

    def body(x_ref, hb_ref):
        xv = x_ref[...]
        mean = jnp.mean(xv, axis=0, keepdims=True)
        var = jnp.mean((xv - mean) ** 2, axis=0, keepdims=True)
        hb_ref[...] = (xv - mean) / jnp.sqrt(var + 1e-5)

    return pl.pallas_call(
        body,
        out_shape=jax.ShapeDtypeStruct((N, D), _f32),
        interpret=interpret,
    )(x)


def _tc_scale(h_bn, deg_raw, *, interpret=False):
    N, D = h_bn.shape
    NP = _pad_rows(N)

    def body(hb_ref, dr_ref, hp_ref, dis_ref):
        dr = dr_ref[...]
        deg = (jnp.sum(dr[:N], axis=1, keepdims=True)
               + jnp.sum(dr[NP:NP + N], axis=1, keepdims=True)) / 8.0
        dis = 1.0 / jnp.sqrt(jnp.maximum(deg, 1.0))
        dis_ref[...] = dis
        hp_ref[...] = hb_ref[...] * dis

    return pl.pallas_call(
        body,
        out_shape=[jax.ShapeDtypeStruct((N, D), _f32),
                   jax.ShapeDtypeStruct((N, 1), _f32)],
        interpret=interpret,
    )(h_bn, deg_raw)


def _tc_mid(agg_raw, dis, W1, b1, W2p, *, interpret=False):
    N = dis.shape[0]
    NP = _pad_rows(N)
    CP = W2p.shape[1]

    def body(ar_ref, dis_ref, w1_ref, b1_ref, w2_ref, u_ref):
        a = ar_ref[...]
        d = dis_ref[...]
        agg = (a[:N] + a[NP:NP + N]) * d
        t = jnp.dot(agg, w1_ref[...], preferred_element_type=_f32,
                    precision=lax.Precision.HIGHEST)
        t = t + b1_ref[...][None, :]
        t = t - jnp.mean(t, axis=0, keepdims=True)
        rn = jnp.sqrt(1e-6 + jnp.mean(jnp.sum(t * t, axis=1)))
        t = jnp.maximum(t / rn, 0.0) * d
        u_ref[...] = jnp.dot(t, w2_ref[...], preferred_element_type=_f32,
                            precision=lax.Precision.HIGHEST)

    return pl.pallas_call(
        body,
        out_shape=jax.ShapeDtypeStruct((N, CP), _f32),
        interpret=interpret,
    )(agg_raw, dis, W1, b1, W2p)


def _tc_out(agg2_raw, dis, b2, *, interpret=False):
    N = dis.shape[0]
    NP = _pad_rows(N)
    C = b2.shape[0]

    def body(ar_ref, dis_ref, b2_ref, o_ref):
        a = ar_ref[...]
        v = (a[:N] + a[NP:NP + N])[:, :C] * dis_ref[...] + b2_ref[...][None, :]
        v = v - jnp.mean(v, axis=0, keepdims=True)
        rn = jnp.sqrt(1e-6 + jnp.mean(jnp.sum(v * v, axis=1)))
        o_ref[...] = v / rn

    return pl.pallas_call(
        body,
        out_shape=jax.ShapeDtypeStruct((N, C), _f32),
        interpret=interpret,
    )(agg2_raw, dis, b2)


def kernel(x, edge_index, W1, b1, W2, b2):
    N, D = x.shape
    E = edge_index.shape[1]
    C = W2.shape[1]
    CP = 48

    ei3 = edge_index.reshape(2, E // _B, _B)

    NP = _pad_rows(N)
    deg_raw = _make_deg(N, E)(
        ei3, jnp.ones((_BD, 8), _f32), jnp.zeros((NP, 8), _f32))
    h_bn = _tc_bn(x)
    h_pre, dis = _tc_scale(h_bn, deg_raw)
    agg1 = _make_agg(N, E, D, _NBUF)(ei3, h_pre, jnp.zeros((NP, D), _f32))
    W2p = jnp.pad(W2, ((0, 0), (0, CP - C)))
    u_pad = _tc_mid(agg1, dis, W1, b1, W2p)
    agg2 = _make_agg(N, E, CP, _NBUF)(ei3, u_pad, jnp.zeros((NP, CP), _f32))
    return _tc_out(agg2, dis, b2)


# speedup vs baseline: 1.1885x; 1.0549x over previous
import functools

import jax
import jax.numpy as jnp
from jax import lax
from jax.experimental import pallas as pl
from jax.experimental.pallas import tpu as pltpu
from jax.experimental.pallas import tpu_sc as plsc

_NC = 2
_NS = 16
_B = 40

_f32 = jnp.float32


def _pad_rows(N):
    return ((N + 8 * _NS - 1) // (8 * _NS)) * (8 * _NS)


_BD = 40


@functools.lru_cache(maxsize=None)
def _make_deg(N, E):
    EPT = E // (_NC * _NS)
    ITERS = EPT // _BD
    GROUPS = ITERS // _NBUF
    NP = _pad_rows(N)
    RPT = NP // _NS
    mesh = plsc.VectorSubcoreMesh(core_axis_name="c", subcore_axis_name="s")

    @functools.partial(
        pl.kernel,
        out_type=jax.ShapeDtypeStruct((_NC * NP, 8), _f32),
        mesh=mesh,
        scratch_types=[
            pltpu.VMEM((ITERS, _BD), jnp.int32),
            pltpu.VMEM((_BD, 8), _f32),
            pltpu.VMEM_SHARED((NP, 8), _f32),
        ] + [pltpu.SemaphoreType.DMA] * _NBUF,
        compiler_params=pltpu.CompilerParams(use_tc_tiling_on_sc=False),
    )
    def deg_kernel(ei3_hbm, ones_hbm, zeros_hbm, out_hbm, idxd, rows, acc,
                   *sems):
        c = lax.axis_index("c")
        s = lax.axis_index("s")
        rowbase = (c * _NS + s) * ITERS
        pltpu.sync_copy(zeros_hbm.at[pl.ds(s * RPT, RPT)],
                        acc.at[pl.ds(s * RPT, RPT)])
        pltpu.sync_copy(ones_hbm, rows)
        pltpu.sync_copy(ei3_hbm.at[1, pl.ds(rowbase, ITERS)], idxd)
        plsc.subcore_barrier()

        for b in range(_NBUF):
            pltpu.async_copy(rows, acc.at[idxd.at[b]], sems[b], add=True)

        def group(g, carry):
            for b in range(_NBUF):
                chunk = g * _NBUF + b
                pltpu.make_async_copy(rows, acc.at[idxd.at[chunk]],
                                      sems[b]).wait()

                @pl.when(g < GROUPS - 1)
                def _next():
                    pltpu.async_copy(rows, acc.at[idxd.at[chunk + _NBUF]],
                                     sems[b], add=True)
            return carry

        lax.fori_loop(0, GROUPS, group, 0)
        plsc.subcore_barrier()
        pltpu.sync_copy(acc.at[pl.ds(s * RPT, RPT)],
                        out_hbm.at[pl.ds(c * NP + s * RPT, RPT)])

    return deg_kernel


_NBUF = 5


@functools.lru_cache(maxsize=None)
def _make_agg(N, E, F, nbuf):
    EPT = E // (_NC * _NS)
    ITERS = EPT // _B
    GROUPS = ITERS // nbuf
    NP = _pad_rows(N)
    RPT = NP // _NS
    mesh = plsc.VectorSubcoreMesh(core_axis_name="c", subcore_axis_name="s")

    @functools.partial(
        pl.kernel,
        out_type=jax.ShapeDtypeStruct((_NC * NP, F), _f32),
        mesh=mesh,
        scratch_types=[
            pltpu.VMEM((ITERS, _B), jnp.int32),
            pltpu.VMEM((ITERS, _B), jnp.int32),
            pltpu.VMEM((nbuf, _B, F), _f32),
            pltpu.VMEM_SHARED((NP, F), _f32),
        ] + [pltpu.SemaphoreType.DMA] * nbuf,
        compiler_params=pltpu.CompilerParams(use_tc_tiling_on_sc=False),
    )
    def agg_kernel(ei3_hbm, table_hbm, zeros_hbm, out_hbm,
                   idxs, idxd, bufs, acc, *sems):
        c = lax.axis_index("c")
        s = lax.axis_index("s")
        rowbase = (c * _NS + s) * ITERS
        pltpu.sync_copy(zeros_hbm.at[pl.ds(s * RPT, RPT)],
                        acc.at[pl.ds(s * RPT, RPT)])
        pltpu.sync_copy(ei3_hbm.at[0, pl.ds(rowbase, ITERS)], idxs)
        pltpu.sync_copy(ei3_hbm.at[1, pl.ds(rowbase, ITERS)], idxd)
        plsc.subcore_barrier()

        for b in range(nbuf):
            pltpu.async_copy(table_hbm.at[idxs.at[b]], bufs.at[b], sems[b])

        def group(g, carry):
            for b in range(nbuf):
                chunk = g * nbuf + b
                pltpu.make_async_copy(table_hbm.at[idxs.at[chunk]],
                                      bufs.at[b], sems[b]).wait()
                pltpu.sync_copy(bufs.at[b], acc.at[idxd.at[chunk]],
                                add=True)

                @pl.when(g < GROUPS - 1)
                def _prefetch():
                    pltpu.async_copy(table_hbm.at[idxs.at[chunk + nbuf]],
                                     bufs.at[b], sems[b])
            return carry

        lax.fori_loop(0, GROUPS, group, 0)
        plsc.subcore_barrier()
        pltpu.sync_copy(acc.at[pl.ds(s * RPT, RPT)],
                        out_hbm.at[pl.ds(c * NP + s * RPT, RPT)])

    return agg_kernel


def _tc_bn(x, *, interpret=False):
    N, D = x.shape

    def body(x_ref, hb_ref):
        xv = x_ref[...]
        mean = jnp.mean(xv, axis=0, keepdims=True)
        var = jnp.mean((xv - mean) ** 2, axis=0, keepdims=True)
        hb_ref[...] = (xv - mean) / jnp.sqrt(var + 1e-5)

    return pl.pallas_call(
        body,
        out_shape=jax.ShapeDtypeStruct((N, D), _f32),
        interpret=interpret,
    )(x)


def _tc_scale(h_bn, deg_raw, *, interpret=False):
    N, D = h_bn.shape
    NP = _pad_rows(N)

    def body(hb_ref, dr_ref, hp_ref, dis_ref):
        dr = dr_ref[...]
        deg = (jnp.sum(dr[:N], axis=1, keepdims=True)
               + jnp.sum(dr[NP:NP + N], axis=1, keepdims=True)) / 8.0
        dis = 1.0 / jnp.sqrt(jnp.maximum(deg, 1.0))
        dis_ref[...] = dis
        hp_ref[...] = hb_ref[...] * dis

    return pl.pallas_call(
        body,
        out_shape=[jax.ShapeDtypeStruct((N, D), _f32),
                   jax.ShapeDtypeStruct((N, 1), _f32)],
        interpret=interpret,
    )(h_bn, deg_raw)


def _tc_mid(agg_raw, dis, W1, b1, W2p, *, interpret=False):
    N = dis.shape[0]
    NP = _pad_rows(N)
    CP = W2p.shape[1]

    def body(ar_ref, dis_ref, w1_ref, b1_ref, w2_ref, u_ref):
        a = ar_ref[...]
        d = dis_ref[...]
        agg = (a[:N] + a[NP:NP + N]) * d
        t = jnp.dot(agg, w1_ref[...], preferred_element_type=_f32)
        t = t + b1_ref[...][None, :]
        t = t - jnp.mean(t, axis=0, keepdims=True)
        rn = jnp.sqrt(1e-6 + jnp.mean(jnp.sum(t * t, axis=1)))
        t = jnp.maximum(t / rn, 0.0) * d
        u_ref[...] = jnp.dot(t, w2_ref[...], preferred_element_type=_f32)

    return pl.pallas_call(
        body,
        out_shape=jax.ShapeDtypeStruct((N, CP), _f32),
        interpret=interpret,
    )(agg_raw, dis, W1, b1, W2p)


def _tc_out(agg2_raw, dis, b2, *, interpret=False):
    N = dis.shape[0]
    NP = _pad_rows(N)
    C = b2.shape[0]

    def body(ar_ref, dis_ref, b2_ref, o_ref):
        a = ar_ref[...]
        v = (a[:N] + a[NP:NP + N])[:, :C] * dis_ref[...] + b2_ref[...][None, :]
        v = v - jnp.mean(v, axis=0, keepdims=True)
        rn = jnp.sqrt(1e-6 + jnp.mean(jnp.sum(v * v, axis=1)))
        o_ref[...] = v / rn

    return pl.pallas_call(
        body,
        out_shape=jax.ShapeDtypeStruct((N, C), _f32),
        interpret=interpret,
    )(agg2_raw, dis, b2)


def kernel(x, edge_index, W1, b1, W2, b2):
    N, D = x.shape
    E = edge_index.shape[1]
    C = W2.shape[1]
    CP = 48

    ei3 = edge_index.reshape(2, E // _B, _B)

    NP = _pad_rows(N)
    deg_raw = _make_deg(N, E)(
        ei3, jnp.ones((_BD, 8), _f32), jnp.zeros((NP, 8), _f32))
    h_bn = _tc_bn(x)
    h_pre, dis = _tc_scale(h_bn, deg_raw)
    agg1 = _make_agg(N, E, D, _NBUF)(ei3, h_pre, jnp.zeros((NP, D), _f32))
    W2p = jnp.pad(W2, ((0, 0), (0, CP - C)))
    u_pad = _tc_mid(agg1, dis, W1, b1, W2p)
    agg2 = _make_agg(N, E, CP, _NBUF)(ei3, u_pad, jnp.zeros((NP, CP), _f32))
    return _tc_out(agg2, dis, b2)


# speedup vs baseline: 1.2375x; 1.0413x over previous
import functools

import jax
import jax.numpy as jnp
from jax import lax
from jax.experimental import pallas as pl
from jax.experimental.pallas import tpu as pltpu
from jax.experimental.pallas import tpu_sc as plsc

_NC = 2
_NS = 16
_B = 40

_f32 = jnp.float32


def _pad_rows(N):
    return ((N + 8 * _NS - 1) // (8 * _NS)) * (8 * _NS)


_BD = 40


@functools.lru_cache(maxsize=None)
def _make_deg(N, E):
    EPT = E // (_NC * _NS)
    ITERS = EPT // _BD
    GROUPS = ITERS // _NBUF
    NP = _pad_rows(N)
    RPT = NP // _NS
    mesh = plsc.VectorSubcoreMesh(core_axis_name="c", subcore_axis_name="s")

    @functools.partial(
        pl.kernel,
        out_type=jax.ShapeDtypeStruct((_NC * NP, 8), _f32),
        mesh=mesh,
        scratch_types=[
            pltpu.VMEM((ITERS, _BD), jnp.int32),
            pltpu.VMEM((_BD, 8), _f32),
            pltpu.VMEM_SHARED((NP, 8), _f32),
        ] + [pltpu.SemaphoreType.DMA] * _NBUF,
        compiler_params=pltpu.CompilerParams(use_tc_tiling_on_sc=False),
    )
    def deg_kernel(ei3_hbm, ones_hbm, zeros_hbm, out_hbm, idxd, rows, acc,
                   *sems):
        c = lax.axis_index("c")
        s = lax.axis_index("s")
        rowbase = (c * _NS + s) * ITERS
        pltpu.sync_copy(zeros_hbm.at[pl.ds(s * RPT, RPT)],
                        acc.at[pl.ds(s * RPT, RPT)])
        pltpu.sync_copy(ones_hbm, rows)
        pltpu.sync_copy(ei3_hbm.at[1, pl.ds(rowbase, ITERS)], idxd)
        plsc.subcore_barrier()

        for b in range(_NBUF):
            pltpu.async_copy(rows, acc.at[idxd.at[b]], sems[b], add=True)

        def group(g, carry):
            for b in range(_NBUF):
                chunk = g * _NBUF + b
                pltpu.make_async_copy(rows, acc.at[idxd.at[chunk]],
                                      sems[b]).wait()

                @pl.when(g < GROUPS - 1)
                def _next():
                    pltpu.async_copy(rows, acc.at[idxd.at[chunk + _NBUF]],
                                     sems[b], add=True)
            return carry

        lax.fori_loop(0, GROUPS, group, 0)
        plsc.subcore_barrier()
        pltpu.sync_copy(acc.at[pl.ds(s * RPT, RPT)],
                        out_hbm.at[pl.ds(c * NP + s * RPT, RPT)])

    return deg_kernel


_NBUF = 5


@functools.lru_cache(maxsize=None)
def _make_agg(N, E, F, nbuf):
    EPT = E // (_NC * _NS)
    ITERS = EPT // _B
    GROUPS = ITERS // nbuf
    NP = _pad_rows(N)
    RPT = NP // _NS
    mesh = plsc.VectorSubcoreMesh(core_axis_name="c", subcore_axis_name="s")

    @functools.partial(
        pl.kernel,
        out_type=jax.ShapeDtypeStruct((_NC * NP, F), _f32),
        mesh=mesh,
        scratch_types=[
            pltpu.VMEM((ITERS, _B), jnp.int32),
            pltpu.VMEM((ITERS, _B), jnp.int32),
            pltpu.VMEM((nbuf, _B, F), _f32),
            pltpu.VMEM_SHARED((NP, F), _f32),
        ] + [pltpu.SemaphoreType.DMA] * nbuf,
        compiler_params=pltpu.CompilerParams(use_tc_tiling_on_sc=False),
    )
    def agg_kernel(ei3_hbm, table_hbm, zeros_hbm, out_hbm,
                   idxs, idxd, bufs, acc, *sems):
        c = lax.axis_index("c")
        s = lax.axis_index("s")
        rowbase = (c * _NS + s) * ITERS
        pltpu.sync_copy(zeros_hbm.at[pl.ds(s * RPT, RPT)],
                        acc.at[pl.ds(s * RPT, RPT)])
        pltpu.sync_copy(ei3_hbm.at[0, pl.ds(rowbase, ITERS)], idxs)
        pltpu.sync_copy(ei3_hbm.at[1, pl.ds(rowbase, ITERS)], idxd)
        plsc.subcore_barrier()

        for b in range(nbuf):
            pltpu.async_copy(table_hbm.at[idxs.at[b]], bufs.at[b], sems[b])

        def group(g, carry):
            for b in range(nbuf):
                chunk = g * nbuf + b
                pltpu.make_async_copy(table_hbm.at[idxs.at[chunk]],
                                      bufs.at[b], sems[b]).wait()
                pltpu.sync_copy(bufs.at[b], acc.at[idxd.at[chunk]],
                                add=True)

                @pl.when(g < GROUPS - 1)
                def _prefetch():
                    pltpu.async_copy(table_hbm.at[idxs.at[chunk + nbuf]],
                                     bufs.at[b], sems[b])
            return carry

        lax.fori_loop(0, GROUPS, group, 0)
        plsc.subcore_barrier()
        pltpu.sync_copy(acc.at[pl.ds(s * RPT, RPT)],
                        out_hbm.at[pl.ds(c * NP + s * RPT, RPT)])

    return agg_kernel


def _tc_bn(x, *, interpret=False):
    N, D = x.shape

    def body(x_ref, hb_ref):
        xv = x_ref[...]
        mean = jnp.mean(xv, axis=0, keepdims=True)
        var = jnp.mean((xv - mean) ** 2, axis=0, keepdims=True)
        hb_ref[...] = (xv - mean) / jnp.sqrt(var + 1e-5)

    return pl.pallas_call(
        body,
        out_shape=jax.ShapeDtypeStruct((N, D), _f32),
        interpret=interpret,
    )(x)


def _tc_scale(h_bn, deg_raw, *, interpret=False):
    N, D = h_bn.shape
    NP = _pad_rows(N)

    def body(hb_ref, dr_ref, hp_ref, dis_ref):
        dr = dr_ref[...]
        deg = (jnp.sum(dr[:N], axis=1, keepdims=True)
               + jnp.sum(dr[NP:NP + N], axis=1, keepdims=True)) / 8.0
        dis = 1.0 / jnp.sqrt(jnp.maximum(deg, 1.0))
        dis_ref[...] = dis
        hp_ref[...] = hb_ref[...] * dis

    return pl.pallas_call(
        body,
        out_shape=[jax.ShapeDtypeStruct((N, D), _f32),
                   jax.ShapeDtypeStruct((N, 1), _f32)],
        interpret=interpret,
    )(h_bn, deg_raw)


def _tc_mid(agg_raw, dis, W1, b1, W2p, *, interpret=False):
    N = dis.shape[0]
    NP = _pad_rows(N)
    CP = W2p.shape[1]

    def body(ar_ref, dis_ref, w1_ref, b1_ref, w2_ref, u_ref):
        a = ar_ref[...]
        d = dis_ref[...]
        agg = (a[:N] + a[NP:NP + N]) * d
        t = jnp.dot(agg, w1_ref[...], preferred_element_type=_f32)
        t = t + b1_ref[...][None, :]
        t = t - jnp.mean(t, axis=0, keepdims=True)
        rn = jnp.sqrt(1e-6 + jnp.mean(jnp.sum(t * t, axis=1)))
        t = jnp.maximum(t / rn, 0.0) * d
        u_ref[...] = jnp.dot(t, w2_ref[...], preferred_element_type=_f32)

    return pl.pallas_call(
        body,
        out_shape=jax.ShapeDtypeStruct((N, CP), _f32),
        interpret=interpret,
    )(agg_raw, dis, W1, b1, W2p)


def _tc_out(agg2_raw, dis, b2, *, interpret=False):
    N = dis.shape[0]
    NP = _pad_rows(N)
    C = b2.shape[0]

    def body(ar_ref, dis_ref, b2_ref, o_ref):
        a = ar_ref[...]
        v = (a[:N] + a[NP:NP + N])[:, :C] * dis_ref[...] + b2_ref[...][None, :]
        v = v - jnp.mean(v, axis=0, keepdims=True)
        rn = jnp.sqrt(1e-6 + jnp.mean(jnp.sum(v * v, axis=1)))
        o_ref[...] = v / rn

    return pl.pallas_call(
        body,
        out_shape=jax.ShapeDtypeStruct((N, C), _f32),
        interpret=interpret,
    )(agg2_raw, dis, b2)


def kernel(x, edge_index, W1, b1, W2, b2):
    N, D = x.shape
    E = edge_index.shape[1]
    C = W2.shape[1]
    CP = 48

    ei3 = edge_index.reshape(2, E // _B, _B)

    NP = _pad_rows(N)
    deg_raw = _make_deg(N, E)(
        ei3, jnp.ones((_BD, 8), _f32), jnp.zeros((NP, 8), _f32))
    h_bn = _tc_bn(x)
    h_pre, dis = _tc_scale(h_bn, deg_raw)
    agg1 = _make_agg(N, E, D, _NBUF)(ei3, h_pre, jnp.zeros((NP, D), _f32))
    W2p = jnp.pad(W2, ((0, 0), (0, CP - C)))
    u_pad = _tc_mid(agg1, dis, W1, b1, W2p)
    agg2 = _make_agg(N, E, CP, 10)(ei3, u_pad, jnp.zeros((NP, CP), _f32))
    return _tc_out(agg2, dis, b2)
